# baseline (speedup 1.0000x reference)
import functools

import jax
import jax.numpy as jnp
from jax import lax
from jax.experimental import pallas as pl
from jax.experimental.pallas import tpu as pltpu
from jax.experimental.pallas import tpu_sc as plsc

B = 8
H, W = 256, 256
N = H * W
NC, NS, L = 2, 16, 16
NW = NC * NS
RCH = 64
NCHUNK = H // RCH
CVECS = W // L
NACC = 4


def _lanesum(vec):
    parts = [vec[i] for i in range(L)]
    while len(parts) > 1:
        parts = [parts[i] + parts[i + 1] for i in range(0, len(parts), 2)]
    return parts[0]


def _worker(pairs, m, smp, out_hbm, w, bufs, sems, outv):
    lane0 = lax.iota(jnp.int32, L) == 0
    zf = jnp.zeros((L,), jnp.float32)

    (lblA, predA), (lblB, predB) = bufs
    (semLA, semPA), (semLB, semPB) = sems
    ref0 = pairs[0][0]

    def start(ci, lbuf, pbuf, sl, sp):
        for mi, (lblr, predr) in enumerate(pairs):
            @pl.when(m == mi)
            def _(lblr=lblr, predr=predr):
                pltpu.async_copy(
                    lblr.at[smp, pl.ds(ci * RCH, RCH), :], lbuf, sl)
                pltpu.async_copy(
                    predr.at[smp, pl.ds(ci * RCH, RCH), :], pbuf, sp)

    def wait(lbuf, pbuf, sl, sp):
        pltpu.make_async_copy(ref0.at[smp, pl.ds(0, RCH), :], lbuf, sl).wait()
        pltpu.make_async_copy(ref0.at[smp, pl.ds(0, RCH), :], pbuf, sp).wait()

    def sweep(row_fn, init):
        start(0, lblA, predA, semLA, semPA)
        start(1, lblB, predB, semLB, semPB)

        def one(ci, lbuf, pbuf, sl, sp, carry, nxt):
            wait(lbuf, pbuf, sl, sp)
            carry = plsc.parallel_loop(0, RCH, carry=carry, unroll=4)(
                lambda r, c: row_fn(r, lbuf, pbuf, c))

            @pl.when(nxt < NCHUNK)
            def _():
                start(nxt, lbuf, pbuf, sl, sp)

            return carry

        def outer(oi, carry):
            ci0 = 2 * oi
            carry = one(ci0, lblA, predA, semLA, semPA, carry, ci0 + 2)
            carry = one(ci0 + 1, lblB, predB, semLB, semPB, carry, ci0 + 3)
            return carry

        return lax.fori_loop(0, NCHUNK // 2, outer, init)

    def acc_row(r, lbuf, pbuf, c2):
        acc = list(c2)
        for c in range(CVECS):
            lb = lbuf[r, pl.ds(c * L, L)]
            pr = pbuf[r, pl.ds(c * L, L)]
            d = pr - lb
            li = d * d
            pos = lb >= 0.1
            s = 3 * (c % NACC)
            acc[s] = acc[s] + jnp.where(pos, 1.0, 0.0)
            acc[s + 1] = acc[s + 1] + jnp.where(pos, li, 0.0)
            acc[s + 2] = acc[s + 2] + li
        return tuple(acc)

    acc = sweep(acc_row, (zf,) * (3 * NACC))
    n_pos = _lanesum(acc[0] + acc[3] + acc[6] + acc[9])
    s_pos = _lanesum(acc[1] + acc[4] + acc[7] + acc[10])
    s_all = _lanesum(acc[2] + acc[5] + acc[8] + acc[11])
    s_neg = s_all - s_pos
    n_neg = jnp.float32(N) - n_pos

    posi_v = jnp.full((L,), s_pos) / jnp.maximum(jnp.full((L,), n_pos), 1.0)
    nega_v = jnp.where(n_neg == 0.0, 0.0,
                       jnp.full((L,), s_neg)
                       / jnp.maximum(jnp.full((L,), n_neg), 1.0))
    outv[...] = jnp.where(lane0, posi_v + nega_v, 0.0)

    rare = (n_pos == 0.0) | (n_neg >= 3.0 * n_pos)

    @pl.when(rare)
    def _rare():
        k_sel = jnp.where(n_pos == 0.0, 500.0,
                          jnp.clip(3.0 * n_pos, 1.0, n_neg))

        def ge_sweep(cand):
            def row(r, lbuf, pbuf, c2):
                cnt_v, sum_v = c2
                for c in range(CVECS):
                    lb = lbuf[r, pl.ds(c * L, L)]
                    pr = pbuf[r, pl.ds(c * L, L)]
                    d = pr - lb
                    a = jnp.where(lb >= 0.1, -1.0, d * d)
                    ge = lax.bitcast_convert_type(a, jnp.int32) >= cand
                    cnt_v = cnt_v + jnp.where(ge, 1.0, 0.0)
                    sum_v = sum_v + jnp.where(ge, a, 0.0)
                return (cnt_v, sum_v)

            cnt_v, sum_v = sweep(row, (zf, zf))
            return _lanesum(cnt_v), _lanesum(sum_v)

        def bit_body(bi, prefix):
            cand = prefix | lax.shift_left(1, 30 - bi)
            cnt, _ = ge_sweep(cand)
            return jnp.where(cnt >= k_sel, cand, prefix)

        prefix = lax.fori_loop(0, 31, bit_body, jnp.int32(0))
        t_f = lax.bitcast_convert_type(prefix, jnp.float32)
        cnt_gt, sum_gt = ge_sweep(prefix + 1)
        sel_sum = sum_gt + t_f * (k_sel - cnt_gt)
        sel_v = jnp.full((L,), sel_sum)
        contrib_v = jnp.where(
            n_pos == 0.0, sel_v / 500.0,
            posi_v + sel_v / jnp.full((L,), k_sel))
        outv[...] = jnp.where(lane0, contrib_v, 0.0)

    pltpu.sync_copy(outv, out_hbm.at[w])


def _sc_body(gh, gah, ox, oy, pgh, pgah, pox, poy, out_hbm,
             lblA, predA, lblB, predB, outv,
             semLA, semPA, semLB, semPB):
    w = lax.axis_index("s") * NC + lax.axis_index("c")
    m = w & 3
    smp = lax.shift_right_logical(w, 2)
    bufs = ((lblA, predA), (lblB, predB))
    sems = ((semLA, semPA), (semLB, semPB))
    pairs = ((gh, pgh), (gah, pgah), (ox, pox), (oy, poy))
    _worker(pairs, m, smp, out_hbm, w, bufs, sems, outv)


@functools.cache
def _sc_call():
    return pl.kernel(
        _sc_body,
        out_type=jax.ShapeDtypeStruct((NW, L), jnp.float32),
        mesh=plsc.VectorSubcoreMesh(core_axis_name="c", subcore_axis_name="s",
                                    num_cores=NC, num_subcores=NS),
        scratch_types=[
            pltpu.VMEM((RCH, W), jnp.float32),
            pltpu.VMEM((RCH, W), jnp.float32),
            pltpu.VMEM((RCH, W), jnp.float32),
            pltpu.VMEM((RCH, W), jnp.float32),
            pltpu.VMEM((L,), jnp.float32),
            pltpu.SemaphoreType.DMA,
            pltpu.SemaphoreType.DMA,
            pltpu.SemaphoreType.DMA,
            pltpu.SemaphoreType.DMA,
        ],
    )


def kernel(gh_label, gah_label, ori_x, ori_y, p_gh, p_gah, p_ori_x, p_ori_y):
    contribs = _sc_call()(gh_label, gah_label, ori_x, ori_y,
                          p_gh, p_gah, p_ori_x, p_ori_y)
    return jnp.sum(contribs) * (1.0 / B)


# speedup vs baseline: 1.0262x; 1.0262x over previous
import functools

import jax
import jax.numpy as jnp
from jax import lax
from jax.experimental import pallas as pl
from jax.experimental.pallas import tpu as pltpu
from jax.experimental.pallas import tpu_sc as plsc

B = 8
H, W = 256, 256
N = H * W
NC, NS, L = 2, 16, 16
NW = NC * NS
RCH = 32
NCHUNK = H // RCH
CVECS = W // L
NACC = 4


def _lanesum(vec):
    parts = [vec[i] for i in range(L)]
    while len(parts) > 1:
        parts = [parts[i] + parts[i + 1] for i in range(0, len(parts), 2)]
    return parts[0]


def _worker(pairs, m, smp, out_hbm, w, bufs, sems, outv):
    lane0 = lax.iota(jnp.int32, L) == 0
    zf = jnp.zeros((L,), jnp.float32)

    (lblA, predA), (lblB, predB) = bufs
    (semLA, semPA), (semLB, semPB) = sems
    ref0 = pairs[0][0]

    def start(ci, lbuf, pbuf, sl, sp):
        for mi, (lblr, predr) in enumerate(pairs):
            @pl.when(m == mi)
            def _(lblr=lblr, predr=predr):
                pltpu.async_copy(
                    lblr.at[smp, pl.ds(ci * RCH, RCH), :], lbuf, sl)
                pltpu.async_copy(
                    predr.at[smp, pl.ds(ci * RCH, RCH), :], pbuf, sp)

    def wait(lbuf, pbuf, sl, sp):
        pltpu.make_async_copy(ref0.at[smp, pl.ds(0, RCH), :], lbuf, sl).wait()
        pltpu.make_async_copy(ref0.at[smp, pl.ds(0, RCH), :], pbuf, sp).wait()

    def sweep(row_fn, init):
        start(0, lblA, predA, semLA, semPA)
        start(1, lblB, predB, semLB, semPB)

        def one(ci, lbuf, pbuf, sl, sp, carry, nxt):
            wait(lbuf, pbuf, sl, sp)
            carry = plsc.parallel_loop(0, RCH, carry=carry, unroll=4)(
                lambda r, c: row_fn(r, lbuf, pbuf, c))

            @pl.when(nxt < NCHUNK)
            def _():
                start(nxt, lbuf, pbuf, sl, sp)

            return carry

        def outer(oi, carry):
            ci0 = 2 * oi
            carry = one(ci0, lblA, predA, semLA, semPA, carry, ci0 + 2)
            carry = one(ci0 + 1, lblB, predB, semLB, semPB, carry, ci0 + 3)
            return carry

        return lax.fori_loop(0, NCHUNK // 2, outer, init)

    def acc_row(r, lbuf, pbuf, c2):
        acc = list(c2)
        for c in range(CVECS):
            lb = lbuf[r, pl.ds(c * L, L)]
            pr = pbuf[r, pl.ds(c * L, L)]
            d = pr - lb
            li = d * d
            pos = lb >= 0.1
            s = 3 * (c % NACC)
            acc[s] = acc[s] + jnp.where(pos, 1.0, 0.0)
            acc[s + 1] = acc[s + 1] + jnp.where(pos, li, 0.0)
            acc[s + 2] = acc[s + 2] + li
        return tuple(acc)

    acc = sweep(acc_row, (zf,) * (3 * NACC))
    n_pos = _lanesum(acc[0] + acc[3] + acc[6] + acc[9])
    s_pos = _lanesum(acc[1] + acc[4] + acc[7] + acc[10])
    s_all = _lanesum(acc[2] + acc[5] + acc[8] + acc[11])
    s_neg = s_all - s_pos
    n_neg = jnp.float32(N) - n_pos

    posi_v = jnp.full((L,), s_pos) / jnp.maximum(jnp.full((L,), n_pos), 1.0)
    nega_v = jnp.where(n_neg == 0.0, 0.0,
                       jnp.full((L,), s_neg)
                       / jnp.maximum(jnp.full((L,), n_neg), 1.0))
    outv[...] = jnp.where(lane0, posi_v + nega_v, 0.0)

    rare = (n_pos == 0.0) | (n_neg >= 3.0 * n_pos)

    @pl.when(rare)
    def _rare():
        k_sel = jnp.where(n_pos == 0.0, 500.0,
                          jnp.clip(3.0 * n_pos, 1.0, n_neg))

        def ge_sweep(cand):
            def row(r, lbuf, pbuf, c2):
                cnt_v, sum_v = c2
                for c in range(CVECS):
                    lb = lbuf[r, pl.ds(c * L, L)]
                    pr = pbuf[r, pl.ds(c * L, L)]
                    d = pr - lb
                    a = jnp.where(lb >= 0.1, -1.0, d * d)
                    ge = lax.bitcast_convert_type(a, jnp.int32) >= cand
                    cnt_v = cnt_v + jnp.where(ge, 1.0, 0.0)
                    sum_v = sum_v + jnp.where(ge, a, 0.0)
                return (cnt_v, sum_v)

            cnt_v, sum_v = sweep(row, (zf, zf))
            return _lanesum(cnt_v), _lanesum(sum_v)

        def bit_body(bi, prefix):
            cand = prefix | lax.shift_left(1, 30 - bi)
            cnt, _ = ge_sweep(cand)
            return jnp.where(cnt >= k_sel, cand, prefix)

        prefix = lax.fori_loop(0, 31, bit_body, jnp.int32(0))
        t_f = lax.bitcast_convert_type(prefix, jnp.float32)
        cnt_gt, sum_gt = ge_sweep(prefix + 1)
        sel_sum = sum_gt + t_f * (k_sel - cnt_gt)
        sel_v = jnp.full((L,), sel_sum)
        contrib_v = jnp.where(
            n_pos == 0.0, sel_v / 500.0,
            posi_v + sel_v / jnp.full((L,), k_sel))
        outv[...] = jnp.where(lane0, contrib_v, 0.0)

    pltpu.sync_copy(outv, out_hbm.at[w])


def _sc_body(gh, gah, ox, oy, pgh, pgah, pox, poy, out_hbm,
             lblA, predA, lblB, predB, outv,
             semLA, semPA, semLB, semPB):
    w = lax.axis_index("s") * NC + lax.axis_index("c")
    m = w & 3
    smp = lax.shift_right_logical(w, 2)
    bufs = ((lblA, predA), (lblB, predB))
    sems = ((semLA, semPA), (semLB, semPB))
    pairs = ((gh, pgh), (gah, pgah), (ox, pox), (oy, poy))
    _worker(pairs, m, smp, out_hbm, w, bufs, sems, outv)


@functools.cache
def _sc_call():
    return pl.kernel(
        _sc_body,
        out_type=jax.ShapeDtypeStruct((NW, L), jnp.float32),
        mesh=plsc.VectorSubcoreMesh(core_axis_name="c", subcore_axis_name="s",
                                    num_cores=NC, num_subcores=NS),
        scratch_types=[
            pltpu.VMEM((RCH, W), jnp.float32),
            pltpu.VMEM((RCH, W), jnp.float32),
            pltpu.VMEM((RCH, W), jnp.float32),
            pltpu.VMEM((RCH, W), jnp.float32),
            pltpu.VMEM((L,), jnp.float32),
            pltpu.SemaphoreType.DMA,
            pltpu.SemaphoreType.DMA,
            pltpu.SemaphoreType.DMA,
            pltpu.SemaphoreType.DMA,
        ],
    )


def kernel(gh_label, gah_label, ori_x, ori_y, p_gh, p_gah, p_ori_x, p_ori_y):
    contribs = _sc_call()(gh_label, gah_label, ori_x, ori_y,
                          p_gh, p_gah, p_ori_x, p_ori_y)
    return jnp.sum(contribs) * (1.0 / B)
